# baseline (speedup 1.0000x reference)
import functools

import jax
import jax.numpy as jnp
import numpy as np
from jax import lax
from jax.experimental import pallas as pl
from jax.experimental.pallas import tpu as pltpu
from jax.experimental.pallas import tpu_sc as plsc

_VOCAB = 30522
_DIM = 768
_MAX_LEN = 64
_BATCH = 4096
_SEQ = 64

_N = _BATCH * _SEQ
_NC = 2
_NS = 16
_NW = _NC * _NS
_SPW = _SEQ // _NW
_RPW = _SPW * _BATCH
_CH = 32
_NBUF = 4
_NCH = _RPW // _CH
_CPP = _BATCH // _CH
_LANES = 16
_COLS = _DIM // _LANES


def _sincos_pos(length, dim):
    pos = np.arange(length, dtype=np.float32)[:, None]
    i = np.arange(dim // 2, dtype=np.float32)[None, :]
    angle = pos / np.power(10000.0, 2.0 * i / dim)
    return np.concatenate([np.sin(angle), np.cos(angle)], axis=-1)


_mesh = plsc.VectorSubcoreMesh(
    core_axis_name="c", subcore_axis_name="s", num_cores=_NC, num_subcores=_NS
)


@functools.partial(
    pl.kernel,
    out_type=jax.ShapeDtypeStruct((_BATCH, _SEQ, _DIM), jnp.float32),
    mesh=_mesh,
    scratch_types=[
        pltpu.VMEM((_RPW,), jnp.int32),
        pltpu.VMEM((_SPW, _DIM), jnp.float32),
        pltpu.VMEM((_NBUF, _CH, _DIM), jnp.float32),
        pltpu.SemaphoreType.DMA((_NBUF,)),
        pltpu.SemaphoreType.DMA((_NBUF,)),
    ],
)
def _embed(xt_hbm, wte_hbm, pos_hbm, out_hbm, idx_v, pos_v, rows_v, gsem, osem):
    wid = lax.axis_index("s") * _NC + lax.axis_index("c")
    s_base = wid * _SPW
    pltpu.sync_copy(xt_hbm.at[pl.ds(s_base * _BATCH, _RPW)], idx_v)
    pltpu.sync_copy(pos_hbm.at[pl.ds(s_base, _SPW)], pos_v)

    def g_desc(c, b):
        return pltpu.make_async_copy(
            wte_hbm.at[idx_v.at[pl.ds(c * _CH, _CH)]],
            rows_v.at[b],
            gsem.at[b],
        )

    def o_desc(c, b):
        phase = c // _CPP
        q0 = (c % _CPP) * _CH
        return pltpu.make_async_copy(
            rows_v.at[b],
            out_hbm.at[pl.ds(q0, _CH), s_base + phase],
            osem.at[b],
        )

    def compute(c, b):
        phase = c // _CPP

        def row(r, _, b=b):
            for cc in range(_COLS):
                sl = pl.ds(cc * _LANES, _LANES)
                p = pos_v[phase, sl]
                plsc.addupdate(rows_v.at[b, r, sl], p)
            return 0

        lax.fori_loop(0, _CH, row, 0)

    def step(c, b, skip_owait=False, issue_ahead=True):
        g_desc(c, b).wait()
        compute(c, b)
        o_desc(c, b).start()
        if issue_ahead:
            f = c + 2
            bf = (b + 2) % _NBUF
            if not skip_owait:
                o_desc(c, bf).wait()
            g_desc(f, bf).start()

    g_desc(0, 0).start()
    g_desc(1, 1).start()

    step(0, 0, skip_owait=True)
    step(1, 1, skip_owait=True)
    step(2, 2)
    step(3, 3)

    def body(i, _):
        for b in range(_NBUF):
            step(_NBUF * i + b, b)
        return 0

    lax.fori_loop(1, _NCH // _NBUF - 1, body, 0)

    step(_NCH - 4, 0)
    step(_NCH - 3, 1)
    step(_NCH - 2, 2, issue_ahead=False)
    step(_NCH - 1, 3, issue_ahead=False)

    for b in range(_NBUF):
        o_desc(_NCH - 4 + b, b).wait()


def kernel(x, wte):
    pos = jnp.asarray(_sincos_pos(_MAX_LEN, _DIM), dtype=jnp.float32)
    xt = jnp.asarray(x, jnp.int32).T.reshape(_SEQ * _BATCH)
    return _embed(xt, wte, pos)


# speedup vs baseline: 1.0200x; 1.0200x over previous
import functools

import jax
import jax.numpy as jnp
import numpy as np
from jax import lax
from jax.experimental import pallas as pl
from jax.experimental.pallas import tpu as pltpu
from jax.experimental.pallas import tpu_sc as plsc

_VOCAB = 30522
_DIM = 768
_MAX_LEN = 64
_BATCH = 4096
_SEQ = 64

_N = _BATCH * _SEQ
_NC = 2
_NS = 16
_NW = _NC * _NS
_RPW = _N // _NW
_CH = 32
_NBUF = 3
_HROWS = _RPW // 2
_HCH = _HROWS // _CH
_LANES = 16
_COLS = _DIM // _LANES


def _sincos_pos(length, dim):
    pos = np.arange(length, dtype=np.float32)[:, None]
    i = np.arange(dim // 2, dtype=np.float32)[None, :]
    angle = pos / np.power(10000.0, 2.0 * i / dim)
    return np.concatenate([np.sin(angle), np.cos(angle)], axis=-1)


_mesh = plsc.VectorSubcoreMesh(
    core_axis_name="c", subcore_axis_name="s", num_cores=_NC, num_subcores=_NS
)


@functools.partial(
    pl.kernel,
    out_type=jax.ShapeDtypeStruct((_N, _DIM), jnp.float32),
    mesh=_mesh,
    scratch_types=[
        pltpu.VMEM((_HROWS,), jnp.int32),
        pltpu.VMEM((_MAX_LEN, _DIM), jnp.float32),
        pltpu.VMEM((_NBUF, _CH, _DIM), jnp.float32),
        pltpu.SemaphoreType.DMA((_NBUF,)),
        pltpu.SemaphoreType.DMA((_NBUF,)),
    ],
)
def _embed(x_hbm, wte_hbm, pos_hbm, out_hbm, idx_v, pos_v, rows_v, gsem, osem):
    wid = lax.axis_index("s") * _NC + lax.axis_index("c")
    base = wid * _RPW
    pltpu.sync_copy(pos_hbm, pos_v)

    def g_desc(c, b):
        return pltpu.make_async_copy(
            wte_hbm.at[idx_v.at[pl.ds(c * _CH, _CH)]],
            rows_v.at[b],
            gsem.at[b],
        )

    def o_desc(rbase, c, b):
        return pltpu.make_async_copy(
            rows_v.at[b],
            out_hbm.at[pl.ds(rbase + c * _CH, _CH)],
            osem.at[b],
        )

    def compute(c, b):
        s0 = lax.rem(c, 2) * _CH

        def row(r, _):
            for cc in range(_COLS):
                sl = pl.ds(cc * _LANES, _LANES)
                p = pos_v[s0 + r, sl]
                plsc.addupdate(rows_v.at[b, r, sl], p)
            return 0

        lax.fori_loop(0, _CH, row, 0)

    def run_pass(rbase):
        def step(c, skip_owait=False, issue_ahead=True):
            b = lax.rem(c, _NBUF)
            g_desc(c, b).wait()
            compute(c, b)
            o_desc(rbase, c, b).start()
            if issue_ahead:
                f = c + 2
                bf = lax.rem(f, _NBUF)
                if not skip_owait:
                    o_desc(rbase, c, bf).wait()
                g_desc(f, bf).start()

        g_desc(0, jnp.int32(0)).start()
        g_desc(1, jnp.int32(1)).start()
        step(jnp.int32(0), skip_owait=True)

        def body(c, _):
            step(c)
            return 0

        lax.fori_loop(1, _HCH - 2, body, 0)

        step(jnp.int32(_HCH - 2), issue_ahead=False)
        step(jnp.int32(_HCH - 1), issue_ahead=False)
        for k in range(_HCH - _NBUF, _HCH):
            o_desc(rbase, jnp.int32(k), jnp.int32(k % _NBUF)).wait()

    pltpu.sync_copy(x_hbm.at[pl.ds(base, _HROWS)], idx_v)
    run_pass(base)
    pltpu.sync_copy(x_hbm.at[pl.ds(base + _HROWS, _HROWS)], idx_v)
    run_pass(base + _HROWS)


def kernel(x, wte):
    pos = jnp.asarray(_sincos_pos(_MAX_LEN, _DIM), dtype=jnp.float32)
    xf = jnp.asarray(x, jnp.int32).reshape(_N)
    out = _embed(xf, wte, pos)
    return out.reshape(_BATCH, _SEQ, _DIM)
